# baseline (speedup 1.0000x reference)
import functools

import jax
import jax.numpy as jnp
from jax import lax
from jax.experimental import pallas as pl
from jax.experimental.pallas import tpu as pltpu
from jax.experimental.pallas import tpu_sc as plsc

B0, B1, N = 32, 32, 32768
ROWS = B0 * B1
L = 16
NC, NS = 2, 16
NW = NC * NS
RPW = ROWS // NW
CHUNKS = N // L
NSETS = 8
UNROLL = 16

_NEG = float("-inf")


_GDN = lax.GatherDimensionNumbers(
    offset_dims=(), collapsed_slice_dims=(0,), start_index_map=(0,))


def _shuf(v, s):
    idx = lax.iota(jnp.int32, L) ^ s
    return lax.gather(v, idx[:, None], _GDN, (1,),
                      mode=lax.GatherScatterMode.PROMISE_IN_BOUNDS)


def _splat_max(v):
    for s in (8, 4, 2, 1):
        v = jnp.maximum(v, _shuf(v, s))
    return v


def _splat_sum(v):
    for s in (8, 4, 2, 1):
        v = v + _shuf(v, s)
    return v


def _vmax_splat(vs):
    m = vs[0]
    for v in vs[1:]:
        m = jnp.maximum(m, v)
    return _splat_max(m)


def _count_eq(vs, mv):
    one = jnp.full((L,), 1.0, jnp.float32)
    zero = jnp.full((L,), 0.0, jnp.float32)
    c = jnp.where(vs[0] == mv, one, zero)
    for v in vs[1:]:
        c = c + jnp.where(v == mv, one, zero)
    return _splat_sum(c)


def _row_threshold_and_scale(buf, lane):
    neg = jnp.full((L,), _NEG, jnp.float32)

    def p1(i, regs):
        regs = list(regs)
        base = i * (L * UNROLL)
        for u in range(UNROLL):
            s = 3 * (u % NSETS)
            x = buf[pl.ds(base + u * L, L)]
            t1, t2, t3 = regs[s], regs[s + 1], regs[s + 2]
            b1 = jnp.maximum(t1, x)
            r1 = jnp.minimum(t1, x)
            b2 = jnp.maximum(t2, r1)
            r2 = jnp.minimum(t2, r1)
            b3 = jnp.maximum(t3, r2)
            regs[s], regs[s + 1], regs[s + 2] = b1, b2, b3
        return tuple(regs)

    cands = list(lax.fori_loop(
        0, CHUNKS // UNROLL, p1, tuple(neg for _ in range(3 * NSETS))))

    m1v = _vmax_splat(cands)
    c1 = _count_eq(cands, m1v)
    mask1 = [jnp.where(v < m1v, v, neg) for v in cands]
    m2v = _vmax_splat(mask1)
    c2 = _count_eq(cands, m2v)
    mask2 = [jnp.where(v < m2v, v, neg) for v in mask1]
    m3v = _vmax_splat(mask2)
    three = jnp.full((L,), 3.0, jnp.float32)
    kv = jnp.where(c1 >= three, m1v,
                   jnp.where(c1 + c2 >= three, m2v, m3v))

    zero = jnp.zeros((L,), jnp.float32)

    one = jnp.full((L,), 1.0, jnp.float32)
    risk = zero
    for s in range(NSETS):
        risk = jnp.maximum(risk, jnp.where(cands[3 * s + 2] == kv, one, zero))
    risky = _splat_max(risk)[0] > 0.0

    def full_sum():
        def p2(i, accs):
            accs = list(accs)
            base = i * (L * UNROLL)
            for u in range(UNROLL):
                x = buf[pl.ds(base + u * L, L)]
                accs[u % NSETS] = accs[u % NSETS] + jnp.where(x >= kv, x, 0.0)
            return tuple(accs)

        accs = lax.fori_loop(0, CHUNKS // UNROLL, p2,
                             tuple(zero for _ in range(NSETS)))
        total = accs[0]
        for a in accs[1:]:
            total = total + a
        lane[0:L] = total

    def cand_sum():
        total = zero
        for v in cands:
            total = total + jnp.where(v >= kv, v, 0.0)
        lane[0:L] = total

    lax.cond(risky, full_sum, cand_sum)
    iv = 1.0 / _splat_sum(lane[0:L])
    return kv, iv


def _mask_normalize(buf, outb, kv, iv):
    def p3(i, c):
        base = i * (L * UNROLL)
        for u in range(UNROLL):
            sl = pl.ds(base + u * L, L)
            x = buf[sl]
            outb[sl] = jnp.where(x >= kv, x * iv, 0.0)
        return c

    lax.fori_loop(0, CHUNKS // UNROLL, p3, 0)


def _sc_body(x_hbm, out_hbm, in0, in1, outb, lane, sem0, sem1, sem_out):
    wid = lax.axis_index("s") * NC + lax.axis_index("c")
    base = wid * RPW
    bufs, sems = (in0, in1), (sem0, sem1)

    pltpu.async_copy(x_hbm.at[base], in0, sem0)
    pltpu.async_copy(x_hbm.at[base + 1], in1, sem1)

    def group(g, c):
        for j in (0, 1):
            r = 2 * g + j
            buf, sem = bufs[j], sems[j]
            pltpu.make_async_copy(x_hbm.at[base + r], buf, sem).wait()
            kv, iv = _row_threshold_and_scale(buf, lane)

            @pl.when(r > 0)
            def _():
                pltpu.make_async_copy(
                    outb, out_hbm.at[base + r - 1], sem_out).wait()

            _mask_normalize(buf, outb, kv, iv)
            pltpu.async_copy(outb, out_hbm.at[base + r], sem_out)

            @pl.when(r + 2 < RPW)
            def _():
                pltpu.async_copy(x_hbm.at[base + r + 2], buf, sem)
        return c

    lax.fori_loop(0, RPW // 2, group, 0)
    pltpu.make_async_copy(outb, out_hbm.at[base + RPW - 1], sem_out).wait()


_mesh = plsc.VectorSubcoreMesh(
    core_axis_name="c", subcore_axis_name="s", num_cores=NC, num_subcores=NS)

_kmax_sc = pl.kernel(
    _sc_body,
    out_type=jax.ShapeDtypeStruct((ROWS, N), jnp.float32),
    mesh=_mesh,
    scratch_types=[
        pltpu.VMEM((N,), jnp.float32),
        pltpu.VMEM((N,), jnp.float32),
        pltpu.VMEM((N,), jnp.float32),
        pltpu.VMEM((L,), jnp.float32),
        pltpu.SemaphoreType.DMA,
        pltpu.SemaphoreType.DMA,
        pltpu.SemaphoreType.DMA,
    ],
)


@jax.jit
def kernel(inputs):
    x = inputs.reshape(ROWS, N)
    out = _kmax_sc(x)
    return out.reshape(B0, B1, N)


# speedup vs baseline: 1.0168x; 1.0168x over previous
import jax
import jax.numpy as jnp
from jax import lax
from jax.experimental import pallas as pl
from jax.experimental.pallas import tpu as pltpu
from jax.experimental.pallas import tpu_sc as plsc

B0, B1, N = 32, 32, 32768
ROWS = B0 * B1
L = 16
NC, NS = 2, 16
NW = NC * NS
RPW = ROWS // NW
CHUNKS = N // L
NSETS = 4
UNROLL = 16

_NEG = float("-inf")


_GDN = lax.GatherDimensionNumbers(
    offset_dims=(), collapsed_slice_dims=(0,), start_index_map=(0,))


def _shuf(v, s):
    idx = lax.iota(jnp.int32, L) ^ s
    return lax.gather(v, idx[:, None], _GDN, (1,),
                      mode=lax.GatherScatterMode.PROMISE_IN_BOUNDS)


def _splat_max(v):
    for s in (8, 4, 2, 1):
        v = jnp.maximum(v, _shuf(v, s))
    return v


def _splat_sum(v):
    for s in (8, 4, 2, 1):
        v = v + _shuf(v, s)
    return v


def _vmax_splat(vs):
    m = vs[0]
    for v in vs[1:]:
        m = jnp.maximum(m, v)
    return _splat_max(m)


def _count_eq(vs, mv):
    one = jnp.full((L,), 1.0, jnp.float32)
    zero = jnp.full((L,), 0.0, jnp.float32)
    c = jnp.where(vs[0] == mv, one, zero)
    for v in vs[1:]:
        c = c + jnp.where(v == mv, one, zero)
    return _splat_sum(c)


def _row_threshold_and_scale(buf, lane):
    neg = jnp.full((L,), _NEG, jnp.float32)

    def p1(i, regs):
        regs = list(regs)
        base = i * (L * UNROLL)
        for u in range(UNROLL):
            s = 3 * (u % NSETS)
            x = buf[pl.ds(base + u * L, L)]
            t1, t2, t3 = regs[s], regs[s + 1], regs[s + 2]
            b1 = jnp.maximum(t1, x)
            r1 = jnp.minimum(t1, x)
            b2 = jnp.maximum(t2, r1)
            r2 = jnp.minimum(t2, r1)
            b3 = jnp.maximum(t3, r2)
            regs[s], regs[s + 1], regs[s + 2] = b1, b2, b3
        return tuple(regs)

    cands = list(lax.fori_loop(
        0, CHUNKS // UNROLL, p1, tuple(neg for _ in range(3 * NSETS))))

    m1v = _vmax_splat(cands)
    c1 = _count_eq(cands, m1v)
    mask1 = [jnp.where(v < m1v, v, neg) for v in cands]
    m2v = _vmax_splat(mask1)
    c2 = _count_eq(cands, m2v)
    mask2 = [jnp.where(v < m2v, v, neg) for v in mask1]
    m3v = _vmax_splat(mask2)
    three = jnp.full((L,), 3.0, jnp.float32)
    kv = jnp.where(c1 >= three, m1v,
                   jnp.where(c1 + c2 >= three, m2v, m3v))

    zero = jnp.zeros((L,), jnp.float32)

    one = jnp.full((L,), 1.0, jnp.float32)
    risk = zero
    for s in range(NSETS):
        risk = jnp.maximum(risk, jnp.where(cands[3 * s + 2] == kv, one, zero))
    risky = _splat_max(risk)[0] > 0.0

    def full_sum():
        def p2(i, accs):
            accs = list(accs)
            base = i * (L * UNROLL)
            for u in range(UNROLL):
                x = buf[pl.ds(base + u * L, L)]
                accs[u % NSETS] = accs[u % NSETS] + jnp.where(x >= kv, x, 0.0)
            return tuple(accs)

        accs = lax.fori_loop(0, CHUNKS // UNROLL, p2,
                             tuple(zero for _ in range(NSETS)))
        total = accs[0]
        for a in accs[1:]:
            total = total + a
        lane[0:L] = total

    def cand_sum():
        total = zero
        for v in cands:
            total = total + jnp.where(v >= kv, v, 0.0)
        lane[0:L] = total

    lax.cond(risky, full_sum, cand_sum)
    iv = 1.0 / _splat_sum(lane[0:L])
    return kv, iv


def _mask_normalize(buf, outb, kv, iv):
    def p3(i, c):
        base = i * (L * UNROLL)
        for u in range(UNROLL):
            sl = pl.ds(base + u * L, L)
            x = buf[sl]
            outb[sl] = jnp.where(x >= kv, x * iv, 0.0)
        return c

    lax.fori_loop(0, CHUNKS // UNROLL, p3, 0)


def _sc_body(x_hbm, out_hbm, in0, in1, outb, lane, sem0, sem1, sem_out):
    wid = lax.axis_index("s") * NC + lax.axis_index("c")
    base = wid * RPW
    bufs, sems = (in0, in1), (sem0, sem1)

    pltpu.async_copy(x_hbm.at[base], in0, sem0)
    pltpu.async_copy(x_hbm.at[base + 1], in1, sem1)

    def group(g, c):
        for j in (0, 1):
            r = 2 * g + j
            buf, sem = bufs[j], sems[j]
            pltpu.make_async_copy(x_hbm.at[base + r], buf, sem).wait()
            kv, iv = _row_threshold_and_scale(buf, lane)

            @pl.when(r > 0)
            def _():
                pltpu.make_async_copy(
                    outb, out_hbm.at[base + r - 1], sem_out).wait()

            _mask_normalize(buf, outb, kv, iv)
            pltpu.async_copy(outb, out_hbm.at[base + r], sem_out)

            @pl.when(r + 2 < RPW)
            def _():
                pltpu.async_copy(x_hbm.at[base + r + 2], buf, sem)
        return c

    lax.fori_loop(0, RPW // 2, group, 0)
    pltpu.make_async_copy(outb, out_hbm.at[base + RPW - 1], sem_out).wait()


_mesh = plsc.VectorSubcoreMesh(
    core_axis_name="c", subcore_axis_name="s", num_cores=NC, num_subcores=NS)

_kmax_sc = pl.kernel(
    _sc_body,
    out_type=jax.ShapeDtypeStruct((ROWS, N), jnp.float32),
    mesh=_mesh,
    scratch_types=[
        pltpu.VMEM((N,), jnp.float32),
        pltpu.VMEM((N,), jnp.float32),
        pltpu.VMEM((N,), jnp.float32),
        pltpu.VMEM((L,), jnp.float32),
        pltpu.SemaphoreType.DMA,
        pltpu.SemaphoreType.DMA,
        pltpu.SemaphoreType.DMA,
    ],
)


@jax.jit
def kernel(inputs):
    x = inputs.reshape(ROWS, N)
    out = _kmax_sc(x)
    return out.reshape(B0, B1, N)


# speedup vs baseline: 1.0172x; 1.0004x over previous
import jax
import jax.numpy as jnp
from jax import lax
from jax.experimental import pallas as pl
from jax.experimental.pallas import tpu as pltpu
from jax.experimental.pallas import tpu_sc as plsc

B0, B1, N = 32, 32, 32768
ROWS = B0 * B1
L = 16
NC, NS = 2, 16
NW = NC * NS
RPW = ROWS // NW
CHUNKS = N // L
NSETS = 4
UNROLL = 16

_NEG = float("-inf")


_GDN = lax.GatherDimensionNumbers(
    offset_dims=(), collapsed_slice_dims=(0,), start_index_map=(0,))


def _shuf(v, s):
    idx = lax.iota(jnp.int32, L) ^ s
    return lax.gather(v, idx[:, None], _GDN, (1,),
                      mode=lax.GatherScatterMode.PROMISE_IN_BOUNDS)


def _splat_max(v):
    for s in (8, 4, 2, 1):
        v = jnp.maximum(v, _shuf(v, s))
    return v


def _splat_sum(v):
    for s in (8, 4, 2, 1):
        v = v + _shuf(v, s)
    return v


def _vmax_splat(vs):
    m = vs[0]
    for v in vs[1:]:
        m = jnp.maximum(m, v)
    return _splat_max(m)


def _count_eq(vs, mv):
    one = jnp.full((L,), 1.0, jnp.float32)
    zero = jnp.full((L,), 0.0, jnp.float32)
    c = jnp.where(vs[0] == mv, one, zero)
    for v in vs[1:]:
        c = c + jnp.where(v == mv, one, zero)
    return _splat_sum(c)


def _row_threshold_and_scale(buf, lane):
    neg = jnp.full((L,), _NEG, jnp.float32)

    def p1(i, regs):
        regs = list(regs)
        base = i * (L * UNROLL)
        for u in range(UNROLL):
            s = 3 * (u % NSETS)
            x = buf[pl.ds(base + u * L, L)]
            t1, t2, t3 = regs[s], regs[s + 1], regs[s + 2]
            b1 = jnp.maximum(t1, x)
            r1 = jnp.minimum(t1, x)
            b2 = jnp.maximum(t2, r1)
            r2 = jnp.minimum(t2, r1)
            b3 = jnp.maximum(t3, r2)
            regs[s], regs[s + 1], regs[s + 2] = b1, b2, b3
        return tuple(regs)

    cands = list(lax.fori_loop(
        0, CHUNKS // UNROLL, p1, tuple(neg for _ in range(3 * NSETS))))

    m1v = _vmax_splat(cands)
    c1 = _count_eq(cands, m1v)
    mask1 = [jnp.where(v < m1v, v, neg) for v in cands]
    m2v = _vmax_splat(mask1)
    c2 = _count_eq(cands, m2v)
    mask2 = [jnp.where(v < m2v, v, neg) for v in mask1]
    m3v = _vmax_splat(mask2)
    three = jnp.full((L,), 3.0, jnp.float32)
    kv = jnp.where(c1 >= three, m1v,
                   jnp.where(c1 + c2 >= three, m2v, m3v))

    zero = jnp.zeros((L,), jnp.float32)

    one = jnp.full((L,), 1.0, jnp.float32)
    risk = zero
    for s in range(NSETS):
        risk = jnp.maximum(risk, jnp.where(cands[3 * s + 2] == kv, one, zero))
    risky = _splat_max(risk)[0] > 0.0

    def full_sum():
        def p2(i, accs):
            accs = list(accs)
            base = i * (L * UNROLL)
            for u in range(UNROLL):
                x = buf[pl.ds(base + u * L, L)]
                accs[u % NSETS] = accs[u % NSETS] + jnp.where(x >= kv, x, 0.0)
            return tuple(accs)

        accs = lax.fori_loop(0, CHUNKS // UNROLL, p2,
                             tuple(zero for _ in range(NSETS)))
        total = accs[0]
        for a in accs[1:]:
            total = total + a
        lane[0:L] = total

    def cand_sum():
        total = zero
        for v in cands:
            total = total + jnp.where(v >= kv, v, 0.0)
        lane[0:L] = total

    lax.cond(risky, full_sum, cand_sum)
    iv = 1.0 / _splat_sum(lane[0:L])
    return kv, iv


def _mask_normalize(buf, outb, kv, iv):
    def p3(i, c):
        base = i * (L * UNROLL)
        for u in range(UNROLL):
            sl = pl.ds(base + u * L, L)
            x = buf[sl]
            outb[sl] = jnp.where(x >= kv, x * iv, 0.0)
        return c

    lax.fori_loop(0, CHUNKS // UNROLL, p3, 0)


def _sc_body(x_hbm, out_hbm, in0, in1, outb, lane, sem0, sem1, sem_out):
    wid = lax.axis_index("s") * NC + lax.axis_index("c")
    base = wid * RPW
    bufs, sems = (in0, in1), (sem0, sem1)

    pltpu.async_copy(x_hbm.at[base], in0, sem0)
    pltpu.async_copy(x_hbm.at[base + 1], in1, sem1)

    def group(g, c):
        for j in (0, 1):
            r = 2 * g + j
            buf, sem = bufs[j], sems[j]
            pltpu.make_async_copy(x_hbm.at[base + r], buf, sem).wait()
            kv, iv = _row_threshold_and_scale(buf, lane)

            @pl.when(r > 0)
            def _():
                pltpu.make_async_copy(
                    outb, out_hbm.at[base + r - 1], sem_out).wait()

            _mask_normalize(buf, outb, kv, iv)
            pltpu.async_copy(outb, out_hbm.at[base + r], sem_out)

            @pl.when(r + 2 < RPW)
            def _():
                pltpu.async_copy(x_hbm.at[base + r + 2], buf, sem)
        return c

    lax.fori_loop(0, RPW // 2, group, 0)
    pltpu.make_async_copy(outb, out_hbm.at[base + RPW - 1], sem_out).wait()


def _make_kmax_sc():
    mesh = plsc.VectorSubcoreMesh(
        core_axis_name="c", subcore_axis_name="s",
        num_cores=NC, num_subcores=NS)
    return pl.kernel(
        _sc_body,
        out_type=jax.ShapeDtypeStruct((ROWS, N), jnp.float32),
        mesh=mesh,
        scratch_types=[
            pltpu.VMEM((N,), jnp.float32),
            pltpu.VMEM((N,), jnp.float32),
            pltpu.VMEM((N,), jnp.float32),
            pltpu.VMEM((L,), jnp.float32),
            pltpu.SemaphoreType.DMA,
            pltpu.SemaphoreType.DMA,
            pltpu.SemaphoreType.DMA,
        ],
    )


@jax.jit
def kernel(inputs):
    x = inputs.reshape(ROWS, N)
    out = _make_kmax_sc()(x)
    return out.reshape(B0, B1, N)
